# baseline (speedup 1.0000x reference)
import functools

import jax
import jax.numpy as jnp
from jax import lax
from jax.experimental import pallas as pl
from jax.experimental.pallas import tpu as pltpu
from jax.experimental.pallas import tpu_sc as plsc

BATCH = 4096
SEQ = 200
D = 32
NC, NS = 2, 16
NW = NC * NS
COLS = BATCH // NW
GROUPS = 250000

_mesh = plsc.VectorSubcoreMesh(core_axis_name="c", subcore_axis_name="s")


@functools.partial(
    pl.kernel,
    mesh=_mesh,
    out_type=jax.ShapeDtypeStruct((SEQ, D, BATCH), jnp.float32),
    scratch_types=[
        pltpu.VMEM((COLS,), jnp.int32),
        pltpu.VMEM((COLS,), jnp.int32),
        pltpu.VMEM((COLS, 128), jnp.float32),
        pltpu.VMEM((D, COLS), jnp.float32),
        pltpu.VMEM((D, 16), jnp.float32),
        pltpu.SemaphoreType.DMA,
    ],
    compiler_params=pltpu.CompilerParams(needs_layout_passes=False),
)
def _emb_lookup(xt_hbm, item_hbm, pos_hbm, out_hbm, xb, qxb, gbuf, obuf,
                psb, sem):
    wid = lax.axis_index("s") * NC + lax.axis_index("c")
    col0 = pl.multiple_of(wid * COLS, COLS)
    iota = lax.iota(jnp.int32, 16)

    def l_body(l, carry):
        pltpu.sync_copy(xt_hbm.at[l, pl.ds(col0, COLS)], xb)
        for m in range(COLS // 16):
            v = xb[pl.ds(16 * m, 16)]
            qxb[pl.ds(16 * m, 16)] = jax.lax.shift_right_logical(v, 2)
        cp = pltpu.async_copy(item_hbm.at[qxb], gbuf, sem)
        pltpu.sync_copy(pos_hbm.at[l], psb)
        cp.wait()
        for m in range(COLS // 16):
            xv = xb[pl.ds(16 * m, 16)]
            rxv = (xv & 3) * D
            rowv = iota + (16 * m)
            for j in range(D):
                g = plsc.load_gather(gbuf, [rowv, rxv + j])
                obuf[j, pl.ds(16 * m, 16)] = g + psb[j, pl.ds(0, 16)]
        pltpu.sync_copy(obuf, out_hbm.at[l, :, pl.ds(col0, COLS)])
        return carry

    lax.fori_loop(0, SEQ, l_body, 0)


def kernel(x, item_emb, pos_emb):
    xt = jnp.swapaxes(x, 0, 1)
    qitem = item_emb[:4 * GROUPS].reshape(GROUPS, 128)
    pos_splat = jnp.broadcast_to(pos_emb[:, :, None], (SEQ, D, 16))
    out_t = _emb_lookup(xt, qitem, pos_splat)
    return lax.transpose(out_t, (2, 0, 1))


# speedup vs baseline: 1.1403x; 1.1403x over previous
import functools

import jax
import jax.numpy as jnp
from jax import lax
from jax.experimental import pallas as pl
from jax.experimental.pallas import tpu as pltpu
from jax.experimental.pallas import tpu_sc as plsc

BATCH = 4096
SEQ = 200
D = 32
NC, NS = 2, 16
NW = NC * NS
COLS = BATCH // NW
GROUPS = 250000
NB = 2

_mesh = plsc.VectorSubcoreMesh(core_axis_name="c", subcore_axis_name="s")


@functools.partial(
    pl.kernel,
    mesh=_mesh,
    out_type=jax.ShapeDtypeStruct((SEQ, D, BATCH), jnp.float32),
    scratch_types=[
        pltpu.VMEM((NB, COLS), jnp.int32),
        pltpu.VMEM((NB, COLS), jnp.int32),
        pltpu.VMEM((NB, COLS, 128), jnp.float32),
        pltpu.VMEM((NB, D, COLS), jnp.float32),
        pltpu.VMEM((SEQ, D), jnp.float32),
        pltpu.SemaphoreType.DMA,
        pltpu.SemaphoreType.DMA,
        pltpu.SemaphoreType.DMA,
        pltpu.SemaphoreType.DMA,
    ],
    compiler_params=pltpu.CompilerParams(needs_layout_passes=False),
)
def _emb_lookup(xt_hbm, item_hbm, pos_hbm, out_hbm, xb, qxb, gbuf, obuf,
                pos_v, isem, gsem, osem0, osem1):
    osems = (osem0, osem1)
    wid = lax.axis_index("s") * NC + lax.axis_index("c")
    col0 = pl.multiple_of(wid * COLS, COLS)
    iota = lax.iota(jnp.int32, 16)
    pltpu.sync_copy(pos_hbm, pos_v)

    def stage(l, s):
        return pltpu.async_copy(xt_hbm.at[l, pl.ds(col0, COLS)], xb.at[s],
                                isem)

    def fire_gather(s):
        for m in range(COLS // 16):
            v = xb[s, pl.ds(16 * m, 16)]
            qxb[s, pl.ds(16 * m, 16)] = jax.lax.shift_right_logical(v, 2)
        return pltpu.async_copy(item_hbm.at[qxb.at[s]], gbuf.at[s], gsem)

    def out_copy(l, s, start=True):
        mk = pltpu.async_copy if start else pltpu.make_async_copy
        return mk(obuf.at[s], out_hbm.at[l, :, pl.ds(col0, COLS)], osems[s])

    def extract(l, s):
        p0 = pos_v[l, pl.ds(0, 16)]
        p1 = pos_v[l, pl.ds(16, 16)]
        for m in range(COLS // 16):
            xv = xb[s, pl.ds(16 * m, 16)]
            rxv = (xv & 3) * D
            rowv = iota + (16 * m)
            for j in range(D):
                src = p0 if j < 16 else p1
                ps = src.at[jnp.full((16,), j % 16, jnp.int32)].get(
                    mode="promise_in_bounds")
                g = plsc.load_gather(gbuf.at[s], [rowv, rxv + j])
                obuf[s, j, pl.ds(16 * m, 16)] = g + ps

    stage(0, 0).wait()
    fire_gather(0)
    stage(1, 1)

    def l_body(l0, carry):
        for b in range(NB):
            l = l0 + b
            s, sn = b, 1 - b
            @pl.when(l + 1 < SEQ)
            def _():
                pltpu.make_async_copy(xt_hbm.at[l, pl.ds(col0, COLS)],
                                      xb.at[sn], isem).wait()
                fire_gather(sn)
            pltpu.make_async_copy(item_hbm.at[qxb.at[s]], gbuf.at[s],
                                  gsem).wait()
            @pl.when(l >= NB)
            def _():
                out_copy(l, s, start=False).wait()
            extract(l, s)
            out_copy(l, s)
            @pl.when(l + 2 < SEQ)
            def _():
                stage(l + 2, s)
        return carry

    lax.fori_loop(0, SEQ // NB, lambda i, c: l_body(i * NB, c), 0)
    for s in range(NB):
        pltpu.make_async_copy(obuf.at[s],
                              out_hbm.at[s, :, pl.ds(col0, COLS)],
                              osems[s]).wait()


def kernel(x, item_emb, pos_emb):
    xt = jnp.swapaxes(x, 0, 1)
    item_t = jnp.swapaxes(item_emb, 0, 1)
    qitem = lax.reshape(item_t[:, :4 * GROUPS], (GROUPS, 128),
                        dimensions=(1, 0))
    out_t = _emb_lookup(xt, qitem, pos_emb)
    return lax.transpose(out_t, (2, 0, 1))


# speedup vs baseline: 1.1802x; 1.0350x over previous
import functools

import jax
import jax.numpy as jnp
from jax import lax
from jax.experimental import pallas as pl
from jax.experimental.pallas import tpu as pltpu
from jax.experimental.pallas import tpu_sc as plsc

BATCH = 4096
SEQ = 200
D = 32
NC, NS = 2, 16
NW = NC * NS
COLS = BATCH // NW
NB = 4

_mesh = plsc.VectorSubcoreMesh(core_axis_name="c", subcore_axis_name="s")


@functools.partial(
    pl.kernel,
    mesh=_mesh,
    out_type=jax.ShapeDtypeStruct((SEQ, D, BATCH), jnp.float32),
    scratch_types=[
        pltpu.VMEM((NB, COLS), jnp.int32),
        pltpu.VMEM((NB, COLS), jnp.int32),
        pltpu.VMEM((NB, COLS, 128), jnp.float32),
        pltpu.VMEM((NB, D, COLS), jnp.float32),
        pltpu.VMEM((SEQ, D), jnp.float32),
        pltpu.SemaphoreType.DMA,
        pltpu.SemaphoreType.DMA,
        pltpu.SemaphoreType.DMA,
        pltpu.SemaphoreType.DMA,
        pltpu.SemaphoreType.DMA,
        pltpu.SemaphoreType.DMA,
    ],
    compiler_params=pltpu.CompilerParams(needs_layout_passes=False),
)
def _emb_lookup(xt_hbm, item_hbm, pos_hbm, out_hbm, xb, qxb, gbuf, obuf,
                pos_v, isem, gsem, os0, os1, os2, os3):
    osems = (os0, os1, os2, os3)
    wid = lax.axis_index("s") * NC + lax.axis_index("c")
    col0 = pl.multiple_of(wid * COLS, COLS)
    iota = lax.iota(jnp.int32, 16)
    pltpu.sync_copy(pos_hbm, pos_v)

    def stage(l, s):
        return pltpu.async_copy(xt_hbm.at[l, pl.ds(col0, COLS)], xb.at[s],
                                isem)

    def wait_stage(l, s):
        pltpu.make_async_copy(xt_hbm.at[l, pl.ds(col0, COLS)], xb.at[s],
                              isem).wait()

    def fire_gather(s):
        for m in range(COLS // 16):
            v = xb[s, pl.ds(16 * m, 16)]
            qxb[s, pl.ds(16 * m, 16)] = jax.lax.shift_right_logical(v, 2)
        return pltpu.async_copy(item_hbm.at[qxb.at[s]], gbuf.at[s], gsem)

    def wait_gather(s):
        pltpu.make_async_copy(item_hbm.at[qxb.at[s]], gbuf.at[s],
                              gsem).wait()

    def out_copy(l, s, start=True):
        mk = pltpu.async_copy if start else pltpu.make_async_copy
        return mk(obuf.at[s], out_hbm.at[l, :, pl.ds(col0, COLS)], osems[s])

    def extract(l, s):
        p0 = pos_v[l, pl.ds(0, 16)]
        p1 = pos_v[l, pl.ds(16, 16)]
        for m in range(COLS // 16):
            xv = xb[s, pl.ds(16 * m, 16)]
            rxv = (xv & 3) * D
            rowv = iota + (16 * m)
            for j in range(D):
                psrc = p0 if j < 16 else p1
                ps = psrc.at[jnp.full((16,), j % 16, jnp.int32)].get(
                    mode="promise_in_bounds")
                g = plsc.load_gather(gbuf.at[s], [rowv, rxv + j])
                obuf[s, j, pl.ds(16 * m, 16)] = g + ps

    stage(0, 0).wait()
    fire_gather(0)
    for b in range(1, NB - 1):
        stage(b, b).wait()
        fire_gather(b)
    stage(NB - 1, NB - 1)

    def l_body(l0, carry):
        for b in range(NB):
            l = l0 + b
            sn = (b + NB - 1) % NB
            @pl.when(l + NB - 1 < SEQ)
            def _():
                wait_stage(l, sn)
                fire_gather(sn)
            wait_gather(b)
            @pl.when(l >= NB)
            def _():
                out_copy(l, b, start=False).wait()
            extract(l, b)
            out_copy(l, b)
            @pl.when(l + NB < SEQ)
            def _():
                stage(l + NB, b)
        return carry

    lax.fori_loop(0, SEQ // NB, lambda i, c: l_body(i * NB, c), 0)
    for s in range(NB):
        pltpu.make_async_copy(obuf.at[s],
                              out_hbm.at[s, :, pl.ds(col0, COLS)],
                              osems[s]).wait()


GROUPS = 250000


def kernel(x, item_emb, pos_emb):
    xt = jnp.swapaxes(x, 0, 1)
    qitem = item_emb[:4 * GROUPS].reshape(GROUPS, 128)
    out_t = _emb_lookup(xt, qitem, pos_emb)
    return lax.transpose(out_t, (2, 0, 1))


# speedup vs baseline: 1.1817x; 1.0013x over previous
import functools

import jax
import jax.numpy as jnp
from jax import lax
from jax.experimental import pallas as pl
from jax.experimental.pallas import tpu as pltpu
from jax.experimental.pallas import tpu_sc as plsc

BATCH = 4096
SEQ = 200
D = 32
NC, NS = 2, 16
NW = NC * NS
COLS = BATCH // NW
NB = 4

_mesh = plsc.VectorSubcoreMesh(core_axis_name="c", subcore_axis_name="s")


@functools.partial(
    pl.kernel,
    mesh=_mesh,
    out_type=jax.ShapeDtypeStruct((SEQ, D, BATCH), jnp.float32),
    scratch_types=[
        pltpu.VMEM((NB, COLS), jnp.int32),
        pltpu.VMEM((NB, COLS), jnp.int32),
        pltpu.VMEM((NB, COLS, 128), jnp.float32),
        pltpu.VMEM((NB, D, COLS), jnp.float32),
        pltpu.VMEM((SEQ, D), jnp.float32),
        pltpu.SemaphoreType.DMA,
        pltpu.SemaphoreType.DMA,
        pltpu.SemaphoreType.DMA,
        pltpu.SemaphoreType.DMA,
        pltpu.SemaphoreType.DMA,
        pltpu.SemaphoreType.DMA,
    ],
    compiler_params=pltpu.CompilerParams(needs_layout_passes=False),
)
def _emb_lookup(xt_hbm, item_hbm, pos_hbm, out_hbm, xb, qxb, gbuf, obuf,
                pos_v, isem, gsem, os0, os1, os2, os3):
    osems = (os0, os1, os2, os3)
    wid = lax.axis_index("s") * NC + lax.axis_index("c")
    col0 = pl.multiple_of(wid * COLS, COLS)
    iota = lax.iota(jnp.int32, 16)
    pltpu.sync_copy(pos_hbm, pos_v)

    def stage(l, s):
        return pltpu.async_copy(xt_hbm.at[l, pl.ds(col0, COLS)], xb.at[s],
                                isem)

    def wait_stage(l, s):
        pltpu.make_async_copy(xt_hbm.at[l, pl.ds(col0, COLS)], xb.at[s],
                              isem).wait()

    def fire_gather(s):
        for m in range(COLS // 16):
            v = xb[s, pl.ds(16 * m, 16)]
            qxb[s, pl.ds(16 * m, 16)] = jax.lax.shift_right_logical(v, 2)
        return pltpu.async_copy(item_hbm.at[qxb.at[s]], gbuf.at[s], gsem)

    def wait_gather(s):
        pltpu.make_async_copy(item_hbm.at[qxb.at[s]], gbuf.at[s],
                              gsem).wait()

    def out_copy(l, s, start=True):
        mk = pltpu.async_copy if start else pltpu.make_async_copy
        return mk(obuf.at[s], out_hbm.at[l, :, pl.ds(col0, COLS)], osems[s])

    def extract(l, s):
        p0 = pos_v[l, pl.ds(0, 16)]
        p1 = pos_v[l, pl.ds(16, 16)]
        ps_all = [
            (p0 if j < 16 else p1).at[
                jnp.full((16,), j % 16, jnp.int32)].get(
                    mode="promise_in_bounds")
            for j in range(D)
        ]
        for m in range(COLS // 16):
            xv = xb[s, pl.ds(16 * m, 16)]
            rxv = (xv & 3) * D
            rowv = iota + (16 * m)
            for j in range(D):
                g = plsc.load_gather(gbuf.at[s], [rowv, rxv + j])
                obuf[s, j, pl.ds(16 * m, 16)] = g + ps_all[j]

    stage(0, 0).wait()
    fire_gather(0)
    for b in range(1, NB - 1):
        stage(b, b).wait()
        fire_gather(b)
    stage(NB - 1, NB - 1)

    def l_body(l0, carry):
        for b in range(NB):
            l = l0 + b
            sn = (b + NB - 1) % NB
            @pl.when(l + NB - 1 < SEQ)
            def _():
                wait_stage(l, sn)
                fire_gather(sn)
            wait_gather(b)
            @pl.when(l >= NB)
            def _():
                out_copy(l, b, start=False).wait()
            extract(l, b)
            out_copy(l, b)
            @pl.when(l + NB < SEQ)
            def _():
                stage(l + NB, b)
        return carry

    lax.fori_loop(0, SEQ // NB, lambda i, c: l_body(i * NB, c), 0)
    for s in range(NB):
        pltpu.make_async_copy(obuf.at[s],
                              out_hbm.at[s, :, pl.ds(col0, COLS)],
                              osems[s]).wait()


GROUPS = 250000


def kernel(x, item_emb, pos_emb):
    xt = jnp.swapaxes(x, 0, 1)
    qitem = item_emb[:4 * GROUPS].reshape(GROUPS, 128)
    out_t = _emb_lookup(xt, qitem, pos_emb)
    return lax.transpose(out_t, (2, 0, 1))


# speedup vs baseline: 1.4274x; 1.2079x over previous
import functools

import jax
import jax.numpy as jnp
from jax import lax
from jax.experimental import pallas as pl
from jax.experimental.pallas import tpu as pltpu
from jax.experimental.pallas import tpu_sc as plsc

BATCH = 4096
SEQ = 200
D = 32
NC, NS = 2, 16
NW = NC * NS
COLS = BATCH // NW
NB = 4

_mesh = plsc.VectorSubcoreMesh(core_axis_name="c", subcore_axis_name="s")


@functools.partial(
    pl.kernel,
    mesh=_mesh,
    out_type=jax.ShapeDtypeStruct((SEQ, D, BATCH), jnp.float32),
    scratch_types=[
        pltpu.VMEM((NB, COLS), jnp.int32),
        pltpu.VMEM((NB, COLS, 128), jnp.float32),
        pltpu.VMEM((NB, D, COLS), jnp.float32),
        pltpu.VMEM((SEQ, D), jnp.float32),
        pltpu.SemaphoreType.DMA,
        pltpu.SemaphoreType.DMA,
        pltpu.SemaphoreType.DMA,
        pltpu.SemaphoreType.DMA,
        pltpu.SemaphoreType.DMA,
        pltpu.SemaphoreType.DMA,
    ],
    compiler_params=pltpu.CompilerParams(needs_layout_passes=False),
)
def _emb_lookup(xt_hbm, item_hbm, pos_hbm, out_hbm, xb, gbuf, obuf,
                pos_v, isem, gsem, os0, os1, os2, os3):
    osems = (os0, os1, os2, os3)
    wid = lax.axis_index("s") * NC + lax.axis_index("c")
    col0 = pl.multiple_of(wid * COLS, COLS)
    iota = lax.iota(jnp.int32, 16)
    pltpu.sync_copy(pos_hbm, pos_v)

    def stage(l, s):
        return pltpu.async_copy(xt_hbm.at[l, pl.ds(col0, COLS)], xb.at[s],
                                isem)

    def wait_stage(l, s):
        pltpu.make_async_copy(xt_hbm.at[l, pl.ds(col0, COLS)], xb.at[s],
                              isem).wait()

    def fire_gather(s):
        return pltpu.async_copy(item_hbm.at[xb.at[s]], gbuf.at[s], gsem)

    def wait_gather(s):
        pltpu.make_async_copy(item_hbm.at[xb.at[s]], gbuf.at[s],
                              gsem).wait()

    def out_copy(l, s, start=True):
        mk = pltpu.async_copy if start else pltpu.make_async_copy
        return mk(obuf.at[s], out_hbm.at[l, :, pl.ds(col0, COLS)], osems[s])

    def extract(l, s):
        p0 = pos_v[l, pl.ds(0, 16)]
        p1 = pos_v[l, pl.ds(16, 16)]
        for k in range(COLS):
            ck = jnp.full((16,), k, jnp.int32)
            g0 = gbuf[s, k, pl.ds(0, 16)] + p0
            g1 = gbuf[s, k, pl.ds(16, 16)] + p1
            plsc.store_scatter(obuf.at[s], [iota, ck], g0)
            plsc.store_scatter(obuf.at[s], [iota + 16, ck], g1)

    stage(0, 0).wait()
    fire_gather(0)
    for b in range(1, NB - 1):
        stage(b, b).wait()
        fire_gather(b)
    stage(NB - 1, NB - 1)

    def l_body(l0, carry):
        for b in range(NB):
            l = l0 + b
            sn = (b + NB - 1) % NB
            @pl.when(l + NB - 1 < SEQ)
            def _():
                wait_stage(l, sn)
                fire_gather(sn)
            wait_gather(b)
            @pl.when(l >= NB)
            def _():
                out_copy(l, b, start=False).wait()
            extract(l, b)
            out_copy(l, b)
            @pl.when(l + NB < SEQ)
            def _():
                stage(l + NB, b)
        return carry

    lax.fori_loop(0, SEQ // NB, lambda i, c: l_body(i * NB, c), 0)
    for s in range(NB):
        pltpu.make_async_copy(obuf.at[s],
                              out_hbm.at[s, :, pl.ds(col0, COLS)],
                              osems[s]).wait()


def kernel(x, item_emb, pos_emb):
    xt = jnp.swapaxes(x, 0, 1)
    item128 = jnp.pad(item_emb, ((0, 0), (0, 128 - D)))
    out_t = _emb_lookup(xt, item128, pos_emb)
    return lax.transpose(out_t, (2, 0, 1))


# speedup vs baseline: 1.5504x; 1.0862x over previous
import functools

import jax
import jax.numpy as jnp
from jax import lax
from jax.experimental import pallas as pl
from jax.experimental.pallas import tpu as pltpu
from jax.experimental.pallas import tpu_sc as plsc

BATCH = 4096
SEQ = 200
D = 32
NC, NS = 2, 16
NW = NC * NS
BPW = BATCH // NW
CB = 8
NCHUNK = BPW // CB
GW = 40

_mesh = plsc.VectorSubcoreMesh(core_axis_name="c", subcore_axis_name="s")


@functools.partial(
    pl.kernel,
    mesh=_mesh,
    out_type=jax.ShapeDtypeStruct((BATCH, SEQ, D), jnp.float32),
    scratch_types=[
        pltpu.VMEM((CB, SEQ), jnp.int32),
        pltpu.VMEM((CB, SEQ, D), jnp.float32),
        pltpu.VMEM((SEQ, D), jnp.float32),
        pltpu.SemaphoreType.DMA,
    ],
    compiler_params=pltpu.CompilerParams(use_tc_tiling_on_sc=False),
)
def _emb_lookup(x_hbm, item_hbm, pos_hbm, out_hbm, idx_v, rows_v, pos_v, sem):
    wid = lax.axis_index("s") * NC + lax.axis_index("c")
    pltpu.sync_copy(pos_hbm, pos_v)
    base_b = wid * BPW

    def chunk_body(c, carry):
        b0 = pl.multiple_of(base_b + c * CB, CB)
        pltpu.sync_copy(x_hbm.at[pl.ds(b0, CB)], idx_v)
        copies = [
            pltpu.async_copy(item_hbm.at[idx_v.at[b, pl.ds(h * GW, GW)]],
                             rows_v.at[b, pl.ds(h * GW, GW)], sem)
            for b in range(CB)
            for h in range(SEQ // GW)
        ]
        for cp in copies:
            cp.wait()

        def row_body(r, rcarry):
            p0 = pos_v[r, pl.ds(0, 16)]
            p1 = pos_v[r, pl.ds(16, 16)]
            for b in range(CB):
                plsc.addupdate(rows_v.at[b, r, pl.ds(0, 16)], p0)
                plsc.addupdate(rows_v.at[b, r, pl.ds(16, 16)], p1)
            return rcarry

        lax.fori_loop(0, SEQ, row_body, 0)
        pltpu.sync_copy(rows_v, out_hbm.at[pl.ds(b0, CB)])
        return carry

    lax.fori_loop(0, NCHUNK, chunk_body, 0)


def kernel(x, item_emb, pos_emb):
    return _emb_lookup(x, item_emb, pos_emb)


# speedup vs baseline: 1.5764x; 1.0167x over previous
import functools

import jax
import jax.numpy as jnp
from jax import lax
from jax.experimental import pallas as pl
from jax.experimental.pallas import tpu as pltpu
from jax.experimental.pallas import tpu_sc as plsc

BATCH = 4096
SEQ = 200
D = 32
NC, NS = 2, 16
NW = NC * NS
BPW = BATCH // NW
CB = 8
NCHUNK = BPW // CB
GW = 40
NB = 2

_mesh = plsc.VectorSubcoreMesh(core_axis_name="c", subcore_axis_name="s")


@functools.partial(
    pl.kernel,
    mesh=_mesh,
    out_type=jax.ShapeDtypeStruct((BATCH, SEQ, D), jnp.float32),
    scratch_types=[
        pltpu.VMEM((NB, CB, SEQ), jnp.int32),
        pltpu.VMEM((NB, CB, SEQ, D), jnp.float32),
        pltpu.VMEM((SEQ, D), jnp.float32),
        pltpu.SemaphoreType.DMA,
        pltpu.SemaphoreType.DMA,
        pltpu.SemaphoreType.DMA,
    ],
    compiler_params=pltpu.CompilerParams(use_tc_tiling_on_sc=False),
)
def _emb_lookup(x_hbm, item_hbm, pos_hbm, out_hbm, idx_v, rows_v, pos_v,
                gsem, osem0, osem1):
    osems = (osem0, osem1)
    wid = lax.axis_index("s") * NC + lax.axis_index("c")
    pltpu.sync_copy(pos_hbm, pos_v)
    base_b = wid * BPW

    def chunk_b0(c):
        return pl.multiple_of(base_b + c * CB, CB)

    def fire_gathers(c, s):
        pltpu.sync_copy(x_hbm.at[pl.ds(chunk_b0(c), CB)], idx_v.at[s])
        for b in range(CB):
            for h in range(SEQ // GW):
                pltpu.async_copy(
                    item_hbm.at[idx_v.at[s, b, pl.ds(h * GW, GW)]],
                    rows_v.at[s, b, pl.ds(h * GW, GW)], gsem)

    def wait_gathers(s):
        for b in range(CB):
            for h in range(SEQ // GW):
                pltpu.make_async_copy(
                    item_hbm.at[idx_v.at[s, b, pl.ds(h * GW, GW)]],
                    rows_v.at[s, b, pl.ds(h * GW, GW)], gsem).wait()

    def out_copy(c, s, start=True):
        mk = pltpu.async_copy if start else pltpu.make_async_copy
        return mk(rows_v.at[s], out_hbm.at[pl.ds(chunk_b0(c), CB)],
                  osems[s])

    def pos_add(s):
        def row_body(r, rcarry):
            p0 = pos_v[r, pl.ds(0, 16)]
            p1 = pos_v[r, pl.ds(16, 16)]
            for b in range(CB):
                plsc.addupdate(rows_v.at[s, b, r, pl.ds(0, 16)], p0)
                plsc.addupdate(rows_v.at[s, b, r, pl.ds(16, 16)], p1)
            return rcarry
        lax.fori_loop(0, SEQ, row_body, 0)

    fire_gathers(0, 0)

    def outer_body(c0, carry):
        for b in range(NB):
            c = c0 + b
            s, sn = b, 1 - b
            @pl.when((c >= 1) & (c + 1 < NCHUNK))
            def _():
                out_copy(c - 1, sn, start=False).wait()
            @pl.when(c + 1 < NCHUNK)
            def _():
                fire_gathers(c + 1, sn)
            wait_gathers(s)
            pos_add(s)
            out_copy(c, s)
        return carry

    lax.fori_loop(0, NCHUNK // NB, lambda i, c: outer_body(i * NB, c), 0)
    for s in range(NB):
        pltpu.make_async_copy(rows_v.at[s],
                              out_hbm.at[pl.ds(chunk_b0(s), CB)],
                              osems[s]).wait()


def kernel(x, item_emb, pos_emb):
    return _emb_lookup(x, item_emb, pos_emb)
